# baseline (speedup 1.0000x reference)
import jax
import jax.numpy as jnp
from jax.experimental import pallas as pl
from jax.experimental.pallas import tpu as pltpu

_LANE = 128
_VMEM_LIMIT = 60 * 1024 * 1024


def _partial_partition_kernel(x_ref, part_ref):
    @pl.when(pl.program_id(1) == 0)
    def _():
        part_ref[...] = jnp.zeros_like(part_ref)

    e = jnp.exp(x_ref[...])
    part_ref[...] += jnp.sum(e, axis=0, keepdims=True)[None]


def _normalize_kernel(x_ref, part_ref, o_ref):
    total = jnp.sum(part_ref[...], axis=0)
    inv = pl.reciprocal(total, approx=False)
    o_ref[...] = jnp.exp(x_ref[...]) * inv


def _softmax_rowband(x, tile_n):
    N, D = x.shape
    n_tiles = N // tile_n
    half = n_tiles // 2

    cost1 = pl.CostEstimate(
        flops=N * D,
        transcendentals=N * D,
        bytes_accessed=N * D * 4 + 2 * D * 4,
    )
    partial = pl.pallas_call(
        _partial_partition_kernel,
        out_shape=jax.ShapeDtypeStruct((2, 1, D), jnp.float32),
        grid=(2, half),
        in_specs=[pl.BlockSpec((tile_n, D), lambda i, k: (i * half + k, 0))],
        out_specs=pl.BlockSpec((1, 1, D), lambda i, k: (i, 0, 0)),
        compiler_params=pltpu.CompilerParams(
            dimension_semantics=("parallel", "arbitrary"),
            vmem_limit_bytes=_VMEM_LIMIT,
        ),
        cost_estimate=cost1,
    )(x)

    cost2 = pl.CostEstimate(
        flops=2 * N * D,
        transcendentals=N * D,
        bytes_accessed=2 * N * D * 4 + 2 * D * 4,
    )
    return pl.pallas_call(
        _normalize_kernel,
        out_shape=jax.ShapeDtypeStruct((N, D), x.dtype),
        grid=(n_tiles,),
        in_specs=[
            pl.BlockSpec((tile_n, D), lambda k: (k, 0)),
            pl.BlockSpec((2, 1, D), lambda k: (0, 0, 0)),
        ],
        out_specs=pl.BlockSpec((tile_n, D), lambda k: (k, 0)),
        compiler_params=pltpu.CompilerParams(
            dimension_semantics=("parallel",),
            vmem_limit_bytes=_VMEM_LIMIT,
        ),
        cost_estimate=cost2,
    )(x, partial)


def _softmax_colblock_kernel(x_ref, o_ref):
    e = jnp.exp(x_ref[...])
    part = jnp.sum(e, axis=0, keepdims=True)
    o_ref[...] = e * pl.reciprocal(part, approx=False)


def _softmax_single_pass(x):
    N, D = x.shape
    tile_d = D
    for t in range(2048, _LANE - 1, -_LANE):
        if D % t == 0:
            tile_d = t
            break
    return pl.pallas_call(
        _softmax_colblock_kernel,
        out_shape=jax.ShapeDtypeStruct((N, D), x.dtype),
        grid=(D // tile_d,),
        in_specs=[pl.BlockSpec((N, tile_d), lambda j: (0, j))],
        out_specs=pl.BlockSpec((N, tile_d), lambda j: (0, j)),
        compiler_params=pltpu.CompilerParams(
            dimension_semantics=("parallel",),
            vmem_limit_bytes=_VMEM_LIMIT,
        ),
    )(x)


def kernel(x):
    orig_shape = x.shape
    N = orig_shape[0]
    x2 = x.reshape(N, -1) if x.ndim != 2 else x
    D = x2.shape[1]

    tile_n = 32
    row_bytes = D * 4
    if (
        N % tile_n == 0
        and (N // tile_n) % 2 == 0
        and N // tile_n >= 4
        and D % _LANE == 0
        and tile_n * row_bytes * 5 <= _VMEM_LIMIT // 2
    ):
        out = _softmax_rowband(x2, tile_n)
    else:
        out = _softmax_single_pass(x2)
    return out.reshape(orig_shape)


# speedup vs baseline: 1.8731x; 1.8731x over previous
import jax
import jax.numpy as jnp
from jax.experimental import pallas as pl
from jax.experimental.pallas import tpu as pltpu

_LANE = 128
_VMEM_LIMIT = 60 * 1024 * 1024


def _softmax_colblock_kernel(x_ref, o_ref):
    e = jnp.exp(x_ref[...])
    part = jnp.sum(e, axis=0, keepdims=True)
    o_ref[...] = e * pl.reciprocal(part, approx=False)


def kernel(x):
    orig_shape = x.shape
    N = orig_shape[0]
    x2 = x.reshape(N, -1) if x.ndim != 2 else x
    D = x2.shape[1]

    budget = _VMEM_LIMIT - 8 * 1024 * 1024
    per_lane = 5 * N * 4
    tile_d = D
    for t in range(D, _LANE - 1, -_LANE):
        if D % t == 0 and t * per_lane <= budget and max(D // t, 1) >= 4:
            tile_d = t
            break
    else:
        for t in range(D, _LANE - 1, -_LANE):
            if D % t == 0 and t * per_lane <= budget:
                tile_d = t
                break

    cost = pl.CostEstimate(
        flops=2 * N * D,
        transcendentals=N * D,
        bytes_accessed=2 * N * D * x2.dtype.itemsize,
    )
    out = pl.pallas_call(
        _softmax_colblock_kernel,
        out_shape=jax.ShapeDtypeStruct((N, D), x2.dtype),
        grid=(D // tile_d,),
        in_specs=[pl.BlockSpec((N, tile_d), lambda j: (0, j))],
        out_specs=pl.BlockSpec((N, tile_d), lambda j: (0, j)),
        compiler_params=pltpu.CompilerParams(
            dimension_semantics=("parallel",),
            vmem_limit_bytes=_VMEM_LIMIT,
        ),
        cost_estimate=cost,
    )(x2)
    return out.reshape(orig_shape)


# speedup vs baseline: 1.8785x; 1.0029x over previous
import jax
import jax.numpy as jnp
from jax.experimental import pallas as pl
from jax.experimental.pallas import tpu as pltpu

_LANE = 128
_VMEM_LIMIT = 60 * 1024 * 1024


_BAND = 64


def _softmax_colblock_kernel(x_ref, o_ref):
    n = x_ref.shape[0]
    band = _BAND if n % _BAND == 0 else n
    part = None
    for b in range(0, n, band):
        e = jnp.exp(x_ref[b:b + band, :])
        s = jnp.sum(e, axis=0, keepdims=True)
        part = s if part is None else part + s
        o_ref[b:b + band, :] = e
    inv = pl.reciprocal(part, approx=False)
    for b in range(0, n, band):
        o_ref[b:b + band, :] = o_ref[b:b + band, :] * inv


def kernel(x):
    orig_shape = x.shape
    N = orig_shape[0]
    x2 = x.reshape(N, -1) if x.ndim != 2 else x
    D = x2.shape[1]

    budget = _VMEM_LIMIT - 8 * 1024 * 1024
    per_lane = 5 * N * 4
    tile_d = D
    for t in range(D, _LANE - 1, -_LANE):
        if D % t == 0 and t * per_lane <= budget and max(D // t, 1) >= 4:
            tile_d = t
            break
    else:
        for t in range(D, _LANE - 1, -_LANE):
            if D % t == 0 and t * per_lane <= budget:
                tile_d = t
                break

    cost = pl.CostEstimate(
        flops=2 * N * D,
        transcendentals=N * D,
        bytes_accessed=2 * N * D * x2.dtype.itemsize,
    )
    out = pl.pallas_call(
        _softmax_colblock_kernel,
        out_shape=jax.ShapeDtypeStruct((N, D), x2.dtype),
        grid=(D // tile_d,),
        in_specs=[pl.BlockSpec((N, tile_d), lambda j: (0, j))],
        out_specs=pl.BlockSpec((N, tile_d), lambda j: (0, j)),
        compiler_params=pltpu.CompilerParams(
            dimension_semantics=("parallel",),
            vmem_limit_bytes=_VMEM_LIMIT,
        ),
        cost_estimate=cost,
    )(x2)
    return out.reshape(orig_shape)
